# baseline (speedup 1.0000x reference)
import functools
import jax
import jax.numpy as jnp
from jax import lax
from jax.experimental import pallas as pl
from jax.experimental.pallas import tpu as pltpu, tpu_sc as plsc

RADIUS2 = 1.0
SHELL2 = 0.25
NBINS = 16
M = 4
K = NBINS * M
QTILE = 512
NW = 32
L = 16


def _knn_tc_kernel(xyz_ref, q_ref, fea_ref, v_ref, m2_ref,
                   idx_ref, s_ref, g_ref, nq):
    t = pl.program_id(1)
    p = xyz_ref[0]
    q = q_ref[0]
    N = p.shape[1]
    Q = q.shape[1]

    x2p = jnp.sum(p * p, axis=0)
    x2q = jnp.sum(q * q, axis=0)
    inner = lax.dot_general(q, p, (((0,), (0,)), ((), ())),
                            preferred_element_type=jnp.float32)
    dist2 = x2q[:, None] + x2p[None, :] - 2.0 * inner

    dxp = p[0][None, :] - q[0][:, None]
    dyp = p[1][None, :] - q[1][:, None]
    dzp = p[2][None, :] - q[2][:, None]
    grp = ((dxp > 0).astype(jnp.int32) * 8 + (dyp > 0).astype(jnp.int32) * 4
           + (dzp > 0).astype(jnp.int32) * 2
           + (dist2 > SHELL2).astype(jnp.int32))
    grp = jnp.where(dist2 <= RADIUS2, grp, NBINS)

    iota = lax.broadcasted_iota(jnp.int32, (Q, N), 1)
    qi = lax.broadcasted_iota(jnp.int32, (Q, 1), 0)[:, 0] + t * nq

    qd = jnp.minimum(dist2 * jnp.float32(1048576.0),
                     jnp.float32(1048575.0)).astype(jnp.int32)
    ikey = (qd << 11) | iota
    MAXI = jnp.int32(0x7FFFFFFF)
    half = N // 2
    for g in range(NBINS):
        d = jnp.where(grp == g, ikey, MAXI)
        dA = d[:, :half]
        dB = d[:, half:]
        pmin = jnp.minimum(dA, dB)
        pmax = jnp.maximum(dA, dB)
        for r in range(M):
            m = jnp.min(pmin, axis=1)
            hit = m != MAXI
            am = m & 2047
            idx_ref[0, 0, g * M + r, :] = jnp.where(hit, am, qi)
            if r < M - 1:
                eq = pmin == m[:, None]
                pmin = jnp.where(eq, pmax, pmin)
                pmax = jnp.where(eq, MAXI, pmax)

    fq = fea_ref[0, :, pl.ds(t * nq, nq)]
    v = v_ref[...]
    s_ref[0, 0, :] = lax.dot_general(v, fq, (((1,), (0,)), ((), ())),
                                     preferred_element_type=jnp.float32)[0]
    g_ref[0] = lax.dot_general(m2_ref[...], fq, (((1,), (0,)), ((), ())),
                               preferred_element_type=jnp.float32)


def _make_sc_kernel(B, N):
    npw = (B * N) // NW
    tiles_pb = N // npw
    ngrp = npw // L
    mesh = plsc.VectorSubcoreMesh(core_axis_name="c", subcore_axis_name="s")

    @functools.partial(
        pl.kernel, mesh=mesh,
        compiler_params=pltpu.CompilerParams(needs_layout_passes=False),
        out_type=jax.ShapeDtypeStruct((NW, 32 * npw), jnp.float32),
        scratch_types=[
            pltpu.VMEM((32 * N,), jnp.float32),
            pltpu.VMEM((N,), jnp.float32),
            pltpu.VMEM((K * npw,), jnp.int32),
            pltpu.VMEM((K * L,), jnp.float32),
            pltpu.VMEM((K * L,), jnp.float32),
            pltpu.VMEM((32 * L,), jnp.float32),
            pltpu.VMEM((32 * npw,), jnp.float32),
        ],
    )
    def sc_fn(g_hbm, s_hbm, idx_hbm, bias_hbm, out_hbm,
              g_v, s_v, idx_v, sbuf, wbuf, bias_v, out_v):
        wid = lax.axis_index("s") * 2 + lax.axis_index("c")
        b = wid // tiles_pb
        pltpu.sync_copy(g_hbm.at[b], g_v)
        pltpu.sync_copy(s_hbm.at[b], s_v)
        pltpu.sync_copy(idx_hbm.at[wid], idx_v)
        pltpu.sync_copy(bias_hbm, bias_v)

        def group_body(t, carry):
            def p1(k, mx):
                iv = idx_v[pl.ds(k * npw + t * L, L)]
                sv = plsc.load_gather(s_v, [iv])
                sbuf[pl.ds(k * L, L)] = sv
                return jnp.maximum(mx, sv)
            mx = lax.fori_loop(0, K, p1, jnp.full((L,), -jnp.inf, jnp.float32))

            def p2(k, den):
                e = jnp.exp(sbuf[pl.ds(k * L, L)] - mx)
                wbuf[pl.ds(k * L, L)] = e
                return den + e
            den = lax.fori_loop(0, K, p2, jnp.zeros((L,), jnp.float32))
            rcp = 1.0 / den

            for h in range(2):
                def p3(k, accs):
                    iv = idx_v[pl.ds(k * npw + t * L, L)]
                    wv = wbuf[pl.ds(k * L, L)]
                    return tuple(
                        accs[ci] + wv * plsc.load_gather(
                            g_v, [iv + (h * 16 + ci) * N])
                        for ci in range(16))
                accs = lax.fori_loop(
                    0, K, p3,
                    tuple(jnp.zeros((L,), jnp.float32) for _ in range(16)))
                for ci in range(16):
                    c = h * 16 + ci
                    bv = bias_v[pl.ds(c * L, L)]
                    out_v[pl.ds(c * npw + t * L, L)] = jnp.maximum(
                        accs[ci] * rcp + bv, 0.0)
            return carry

        lax.fori_loop(0, ngrp, group_body, 0)
        pltpu.sync_copy(out_v, out_hbm.at[wid])

    return sc_fn, npw, tiles_pb


def kernel(xyz, fea, W, altha, mlp_w, mlp_b):
    B, _, N = xyz.shape
    Q = QTILE
    v = altha @ W
    M2 = mlp_w @ W

    grid = (B, N // Q)
    idx4, s2, g = pl.pallas_call(
        functools.partial(_knn_tc_kernel, nq=Q),
        grid=grid,
        in_specs=[
            pl.BlockSpec((1, 3, N), lambda b, t: (b, 0, 0)),
            pl.BlockSpec((1, 3, Q), lambda b, t: (b, 0, t)),
            pl.BlockSpec((1, 32, N), lambda b, t: (b, 0, 0)),
            pl.BlockSpec((1, 32), lambda b, t: (0, 0)),
            pl.BlockSpec((32, 32), lambda b, t: (0, 0)),
        ],
        out_specs=[
            pl.BlockSpec((1, 1, K, Q), lambda b, t: (b, t, 0, 0)),
            pl.BlockSpec((1, 1, Q), lambda b, t: (b, 0, t)),
            pl.BlockSpec((1, 32, Q), lambda b, t: (b, 0, t)),
        ],
        out_shape=[
            jax.ShapeDtypeStruct((B, N // Q, K, Q), jnp.int32),
            jax.ShapeDtypeStruct((B, 1, N), jnp.float32),
            jax.ShapeDtypeStruct((B, 32, N), jnp.float32),
        ],
    )(xyz, xyz, fea, v, M2)

    sc_fn, npw, tiles_pb = _make_sc_kernel(B, N)
    T = N // Q
    spw = Q // npw
    g_flat = g.reshape(B, 32 * N)
    s_flat = s2.reshape(B, N)
    idx_flat = (idx4.reshape(B, T, K, spw, npw)
                .transpose(0, 1, 3, 2, 4).reshape(NW, K * npw))
    bias_tab = jnp.broadcast_to(mlp_b.reshape(32, 1), (32, L)).reshape(32 * L)
    out_w = sc_fn(g_flat, s_flat, idx_flat, bias_tab)
    out = (out_w.reshape(B, T, spw, 32, npw)
           .transpose(0, 3, 1, 2, 4).reshape(B, 32, N))
    return out


# speedup vs baseline: 1.1432x; 1.1432x over previous
import functools
import jax
import jax.numpy as jnp
from jax import lax
from jax.experimental import pallas as pl
from jax.experimental.pallas import tpu as pltpu, tpu_sc as plsc

RADIUS2 = 1.0
SHELL2 = 0.25
NBINS = 16
M = 4
K = NBINS * M
QTILE = 256
NW = 32
L = 16


def _knn_tc_kernel(xyz_ref, q_ref, fea_ref, v_ref, m2_ref,
                   idx_ref, s_ref, g_ref, nq):
    t = pl.program_id(1)
    p = xyz_ref[0]
    q = q_ref[0]
    N = p.shape[1]
    Q = q.shape[1]

    x2p = jnp.sum(p * p, axis=0)
    x2q = jnp.sum(q * q, axis=0)
    inner = lax.dot_general(q, p, (((0,), (0,)), ((), ())),
                            preferred_element_type=jnp.float32)
    dist2 = x2q[:, None] + x2p[None, :] - 2.0 * inner

    dxp = p[0][None, :] - q[0][:, None]
    dyp = p[1][None, :] - q[1][:, None]
    dzp = p[2][None, :] - q[2][:, None]
    grp = ((dxp > 0).astype(jnp.int32) * 8 + (dyp > 0).astype(jnp.int32) * 4
           + (dzp > 0).astype(jnp.int32) * 2
           + (dist2 > SHELL2).astype(jnp.int32))
    grp = jnp.where(dist2 <= RADIUS2, grp, NBINS)

    iota = lax.broadcasted_iota(jnp.int32, (Q, N), 1)
    qi = lax.broadcasted_iota(jnp.int32, (Q, 1), 0)[:, 0] + t * nq

    qd = jnp.minimum(dist2 * jnp.float32(1048576.0),
                     jnp.float32(1048575.0)).astype(jnp.int32)
    ikey = (qd << 11) | iota
    MAXI = jnp.int32(0x7FFFFFFF)
    half = N // 2
    for g in range(NBINS):
        d = jnp.where(grp == g, ikey, MAXI)
        dA = d[:, :half]
        dB = d[:, half:]
        pmin = jnp.minimum(dA, dB)
        pmax = jnp.maximum(dA, dB)
        for r in range(M):
            m = jnp.min(pmin, axis=1)
            hit = m != MAXI
            am = m & 2047
            idx_ref[0, 0, g * M + r, :] = jnp.where(hit, am, qi)
            if r < M - 1:
                eq = pmin == m[:, None]
                pmin = jnp.where(eq, pmax, pmin)
                pmax = jnp.where(eq, MAXI, pmax)

    fq = fea_ref[0, :, pl.ds(t * nq, nq)]
    v = v_ref[...]
    s_ref[0, 0, :] = lax.dot_general(v, fq, (((1,), (0,)), ((), ())),
                                     preferred_element_type=jnp.float32)[0]
    g_ref[0] = lax.dot_general(m2_ref[...], fq, (((1,), (0,)), ((), ())),
                               preferred_element_type=jnp.float32)


def _make_sc_kernel(B, N):
    npw = (B * N) // NW
    tiles_pb = N // npw
    ngrp = npw // L
    mesh = plsc.VectorSubcoreMesh(core_axis_name="c", subcore_axis_name="s")

    @functools.partial(
        pl.kernel, mesh=mesh,
        compiler_params=pltpu.CompilerParams(needs_layout_passes=False),
        out_type=jax.ShapeDtypeStruct((NW, 32 * npw), jnp.float32),
        scratch_types=[
            pltpu.VMEM((32 * N,), jnp.float32),
            pltpu.VMEM((N,), jnp.float32),
            pltpu.VMEM((K * npw,), jnp.int32),
            pltpu.VMEM((K * L,), jnp.float32),
            pltpu.VMEM((K * L,), jnp.float32),
            pltpu.VMEM((32 * L,), jnp.float32),
            pltpu.VMEM((32 * npw,), jnp.float32),
        ],
    )
    def sc_fn(g_hbm, s_hbm, idx_hbm, bias_hbm, out_hbm,
              g_v, s_v, idx_v, sbuf, wbuf, bias_v, out_v):
        wid = lax.axis_index("s") * 2 + lax.axis_index("c")
        b = wid // tiles_pb
        pltpu.sync_copy(g_hbm.at[b], g_v)
        pltpu.sync_copy(s_hbm.at[b], s_v)
        pltpu.sync_copy(idx_hbm.at[wid], idx_v)
        pltpu.sync_copy(bias_hbm, bias_v)

        def group_body(t, carry):
            def p1(k, mx):
                iv = idx_v[pl.ds(k * npw + t * L, L)]
                sv = plsc.load_gather(s_v, [iv])
                sbuf[pl.ds(k * L, L)] = sv
                return jnp.maximum(mx, sv)
            mx = lax.fori_loop(0, K, p1, jnp.full((L,), -jnp.inf, jnp.float32))

            def p2(k, den):
                e = jnp.exp(sbuf[pl.ds(k * L, L)] - mx)
                wbuf[pl.ds(k * L, L)] = e
                return den + e
            den = lax.fori_loop(0, K, p2, jnp.zeros((L,), jnp.float32))
            rcp = 1.0 / den

            for h in range(2):
                def p3(k, accs):
                    iv = idx_v[pl.ds(k * npw + t * L, L)]
                    wv = wbuf[pl.ds(k * L, L)]
                    return tuple(
                        accs[ci] + wv * plsc.load_gather(
                            g_v, [iv + (h * 16 + ci) * N])
                        for ci in range(16))
                accs = lax.fori_loop(
                    0, K, p3,
                    tuple(jnp.zeros((L,), jnp.float32) for _ in range(16)))
                for ci in range(16):
                    c = h * 16 + ci
                    bv = bias_v[pl.ds(c * L, L)]
                    out_v[pl.ds(c * npw + t * L, L)] = jnp.maximum(
                        accs[ci] * rcp + bv, 0.0)
            return carry

        lax.fori_loop(0, ngrp, group_body, 0)
        pltpu.sync_copy(out_v, out_hbm.at[wid])

    return sc_fn, npw, tiles_pb


def kernel(xyz, fea, W, altha, mlp_w, mlp_b):
    B, _, N = xyz.shape
    Q = QTILE
    v = altha @ W
    M2 = mlp_w @ W

    grid = (B, N // Q)
    idx4, s2, g = pl.pallas_call(
        functools.partial(_knn_tc_kernel, nq=Q),
        grid=grid,
        in_specs=[
            pl.BlockSpec((1, 3, N), lambda b, t: (b, 0, 0)),
            pl.BlockSpec((1, 3, Q), lambda b, t: (b, 0, t)),
            pl.BlockSpec((1, 32, N), lambda b, t: (b, 0, 0)),
            pl.BlockSpec((1, 32), lambda b, t: (0, 0)),
            pl.BlockSpec((32, 32), lambda b, t: (0, 0)),
        ],
        out_specs=[
            pl.BlockSpec((1, 1, K, Q), lambda b, t: (b, t, 0, 0)),
            pl.BlockSpec((1, 1, Q), lambda b, t: (b, 0, t)),
            pl.BlockSpec((1, 32, Q), lambda b, t: (b, 0, t)),
        ],
        out_shape=[
            jax.ShapeDtypeStruct((B, N // Q, K, Q), jnp.int32),
            jax.ShapeDtypeStruct((B, 1, N), jnp.float32),
            jax.ShapeDtypeStruct((B, 32, N), jnp.float32),
        ],
    )(xyz, xyz, fea, v, M2)

    sc_fn, npw, tiles_pb = _make_sc_kernel(B, N)
    g_flat = g.reshape(B, 32 * N)
    s_flat = s2.reshape(B, N)
    idx_flat = idx4.reshape(NW, K * npw)
    bias_tab = jnp.broadcast_to(mlp_b.reshape(32, 1), (32, L)).reshape(32 * L)
    out_w = sc_fn(g_flat, s_flat, idx_flat, bias_tab)
    out = out_w.reshape(B, tiles_pb, 32, npw).transpose(0, 2, 1, 3).reshape(B, 32, N)
    return out
